# baseline (speedup 1.0000x reference)
import functools

import numpy as np
import jax
import jax.numpy as jnp
from jax import lax
from jax.experimental import pallas as pl
from jax.experimental.pallas import tpu as pltpu
from jax.experimental.pallas import tpu_sc as plsc

_D = 10
_HW = 224
_NPIX = 2 * _HW * _HW
_LAYERS = ((96, 56), (192, 28), (384, 14))

_NC, _NS = 2, 16
_NW = _NC * _NS
_CHUNK = _NPIX // _NW
_NV = _CHUNK // 16


def _wmat(in_size):
    c = (np.arange(_HW) + 0.5) * in_size / _HW - 0.5
    w = np.maximum(0.0, 1.0 - np.abs(c[:, None] - np.arange(in_size)[None, :]))
    return (w / w.sum(1, keepdims=True)).astype(np.float32)


_WX = {h: _wmat(h) for _, h in _LAYERS}
_WYT = {h: _wmat(h).T.copy() for _, h in _LAYERS}


def _edges_body(dref, eref):
    x = dref[...]
    mn = jnp.min(x)
    mx = jnp.max(x)
    step = (mx - mn) / np.float32(_D)
    io = lax.broadcasted_iota(jnp.int32, (16, 128), 0).astype(jnp.float32)
    eref[...] = mn + step * io


def _sc_build():
    mesh = plsc.VectorSubcoreMesh(core_axis_name="c", subcore_axis_name="s")

    @functools.partial(
        pl.kernel, mesh=mesh,
        out_type=jax.ShapeDtypeStruct((_NW, _D, 16), jnp.float32),
        scratch_types=[
            pltpu.VMEM((_CHUNK,), jnp.float32),
            pltpu.VMEM((16, 16), jnp.float32),
            pltpu.VMEM((_D, 16), jnp.float32),
        ],
    )
    def sc_kern(dep_hbm, edges_hbm, part_hbm, dvm, evm, cvm):
        wid = lax.axis_index("s") * _NC + lax.axis_index("c")
        base = wid * _CHUNK
        pltpu.sync_copy(dep_hbm.at[pl.ds(base, _CHUNK)], dvm)
        pltpu.sync_copy(edges_hbm, evm)

        ones_f = jnp.ones((16,), jnp.float32)
        zero_f = jnp.zeros((16,), jnp.float32)

        def body(j, accs):
            v = dvm[pl.ds(j * 16, 16)]
            return tuple(acc + jnp.where(v >= evm[k], ones_f, zero_f)
                         for k, acc in zip(range(1, _D), accs))

        accs = lax.fori_loop(
            0, _NV, body,
            tuple(jnp.zeros((16,), jnp.float32) for _ in range(_D - 1)),
            unroll=8)
        cvm[0] = np.float32(_NV) - accs[0]
        for dd in range(1, _D - 1):
            cvm[dd] = accs[dd - 1] - accs[dd]
        cvm[_D - 1] = accs[_D - 2]
        pltpu.sync_copy(cvm, part_hbm.at[wid])

    return sc_kern


def _c1_body(dref, wx1, wyt1, wx2, wyt2, wx3, wyt3, o1, o2, o3):
    d3 = dref[...]
    mn = jnp.min(d3)
    mx = jnp.max(d3)
    step = (mx - mn) / np.float32(_D)
    bi = jnp.zeros(d3.shape, jnp.int32)
    for k in range(1, _D):
        bi = bi + (d3 >= mn + step * np.float32(k)).astype(jnp.int32)
    for b in range(2):
        bib = bi[b]
        tall = jnp.concatenate(
            [(bib == dd).astype(jnp.float32) for dd in range(_D)], axis=0)
        for wxr, wytr, oref in ((wx1, wyt1, o1), (wx2, wyt2, o2),
                                (wx3, wyt3, o3)):
            t1 = jnp.dot(tall, wxr[...])
            wide = jnp.concatenate(
                [t1[dd * _HW:(dd + 1) * _HW, :] for dd in range(_D)], axis=1)
            oref[b] = jnp.dot(wytr[...], wide)


def _c2_body(a1, b1, a2, b2, a3, b3, pref, out):
    cnt = jnp.sum(pref[...], axis=(0, 2))
    inv = 1.0 / jnp.maximum(cnt, np.float32(1e-6))
    out[...] = jnp.zeros((3, 384, _D), jnp.float32)
    out[0, 0:96, :] = jnp.dot(a1[...], b1[...]) * inv
    out[1, 0:192, :] = jnp.dot(a2[...], b2[...]) * inv
    out[2, :, :] = jnp.dot(a3[...], b3[...]) * inv


def kernel(fmap1, fmap2, fmap3, depths):
    d3 = depths.reshape(2, _HW, _HW)
    edges = pl.pallas_call(
        _edges_body,
        out_shape=jax.ShapeDtypeStruct((16, 128), jnp.float32),
    )(depths.reshape(2 * _HW, _HW))

    parts = _sc_build()(depths.reshape(_NPIX), edges[:, :16])

    c1_out = pl.pallas_call(
        _c1_body,
        out_shape=[
            jax.ShapeDtypeStruct((2, 56, _D * 56), jnp.float32),
            jax.ShapeDtypeStruct((2, 28, _D * 28), jnp.float32),
            jax.ShapeDtypeStruct((2, 14, _D * 14), jnp.float32),
        ],
    )(d3,
      jnp.asarray(_WX[56]), jnp.asarray(_WYT[56]),
      jnp.asarray(_WX[28]), jnp.asarray(_WYT[28]),
      jnp.asarray(_WX[14]), jnp.asarray(_WYT[14]))

    mats = []
    for idx, ((c, h), o) in enumerate(zip(_LAYERS, c1_out)):
        f = (fmap1, fmap2, fmap3)[idx]
        a = f.transpose(1, 0, 2, 3).reshape(c, 2 * h * h)
        bm = o.reshape(2, h, _D, h).transpose(0, 1, 3, 2).reshape(2 * h * h, _D)
        mats += [a, bm]

    return pl.pallas_call(
        _c2_body,
        out_shape=jax.ShapeDtypeStruct((3, 384, _D), jnp.float32),
    )(*mats, parts)


# speedup vs baseline: 1.0548x; 1.0548x over previous
import functools

import numpy as np
import jax
import jax.numpy as jnp
from jax import lax
from jax.experimental import pallas as pl
from jax.experimental.pallas import tpu as pltpu
from jax.experimental.pallas import tpu_sc as plsc

_D = 10
_HW = 224
_NPIX = 2 * _HW * _HW
_LAYERS = ((96, 56), (192, 28), (384, 14))

_NC, _NS = 2, 16
_NW = _NC * _NS
_CHUNK = _NPIX // _NW
_NV = _CHUNK // 16


def _wmat(in_size):
    c = (np.arange(_HW) + 0.5) * in_size / _HW - 0.5
    w = np.maximum(0.0, 1.0 - np.abs(c[:, None] - np.arange(in_size)[None, :]))
    return (w / w.sum(1, keepdims=True)).astype(np.float32)


_WX = {h: _wmat(h) for _, h in _LAYERS}
_WYT = {h: _wmat(h).T.copy() for _, h in _LAYERS}


def _sc_build():
    mesh = plsc.VectorSubcoreMesh(core_axis_name="c", subcore_axis_name="s")

    @functools.partial(
        pl.kernel, mesh=mesh,
        out_type=jax.ShapeDtypeStruct((_NW, _D, 16), jnp.float32),
        scratch_types=[
            pltpu.VMEM((_CHUNK,), jnp.float32),
            pltpu.VMEM((16, 16), jnp.float32),
            pltpu.VMEM((_D, 16), jnp.float32),
        ],
    )
    def sc_kern(dep_hbm, edges_hbm, part_hbm, dvm, evm, cvm):
        wid = lax.axis_index("s") * _NC + lax.axis_index("c")
        base = wid * _CHUNK
        pltpu.sync_copy(dep_hbm.at[pl.ds(base, _CHUNK)], dvm)
        pltpu.sync_copy(edges_hbm, evm)

        ones_f = jnp.ones((16,), jnp.float32)
        zero_f = jnp.zeros((16,), jnp.float32)

        def body(j, accs):
            v = dvm[pl.ds(j * 16, 16)]
            return tuple(acc + jnp.where(v >= evm[k], ones_f, zero_f)
                         for k, acc in zip(range(1, _D), accs))

        accs = lax.fori_loop(
            0, _NV, body,
            tuple(jnp.zeros((16,), jnp.float32) for _ in range(_D - 1)),
            unroll=8)
        cvm[0] = np.float32(_NV) - accs[0]
        for dd in range(1, _D - 1):
            cvm[dd] = accs[dd - 1] - accs[dd]
        cvm[_D - 1] = accs[_D - 2]
        pltpu.sync_copy(cvm, part_hbm.at[wid])

    return sc_kern


def _c1_body(dref, wx1, wyt1, wx2, wyt2, wx3, wyt3, o1, o2, o3, eref):
    d3 = dref[...]
    mn = jnp.min(d3)
    mx = jnp.max(d3)
    step = (mx - mn) / np.float32(_D)
    io = lax.broadcasted_iota(jnp.int32, (16, 128), 0).astype(jnp.float32)
    eref[...] = mn + step * io
    bi = jnp.zeros(d3.shape, jnp.int32)
    for k in range(1, _D):
        bi = bi + (d3 >= mn + step * np.float32(k)).astype(jnp.int32)
    for b in range(2):
        bib = bi[b]
        tall = jnp.concatenate(
            [(bib == dd).astype(jnp.float32) for dd in range(_D)], axis=0)
        for wxr, wytr, oref in ((wx1, wyt1, o1), (wx2, wyt2, o2),
                                (wx3, wyt3, o3)):
            t1 = jnp.dot(tall, wxr[...])
            wide = jnp.concatenate(
                [t1[dd * _HW:(dd + 1) * _HW, :] for dd in range(_D)], axis=1)
            oref[b] = jnp.dot(wytr[...], wide)


def _c2_body(a1, b1, a2, b2, a3, b3, pref, out):
    cnt = jnp.sum(pref[...], axis=(0, 2))
    inv = 1.0 / jnp.maximum(cnt, np.float32(1e-6))
    out[...] = jnp.zeros((3, 384, _D), jnp.float32)
    out[0, 0:96, :] = (jnp.dot(a1[0], b1[0]) + jnp.dot(a1[1], b1[1])) * inv
    out[1, 0:192, :] = (jnp.dot(a2[0], b2[0]) + jnp.dot(a2[1], b2[1])) * inv
    out[2, :, :] = (jnp.dot(a3[0], b3[0]) + jnp.dot(a3[1], b3[1])) * inv


def kernel(fmap1, fmap2, fmap3, depths):
    d3 = depths.reshape(2, _HW, _HW)
    c1_out = pl.pallas_call(
        _c1_body,
        out_shape=[
            jax.ShapeDtypeStruct((2, 56, _D * 56), jnp.float32),
            jax.ShapeDtypeStruct((2, 28, _D * 28), jnp.float32),
            jax.ShapeDtypeStruct((2, 14, _D * 14), jnp.float32),
            jax.ShapeDtypeStruct((16, 128), jnp.float32),
        ],
    )(d3,
      jnp.asarray(_WX[56]), jnp.asarray(_WYT[56]),
      jnp.asarray(_WX[28]), jnp.asarray(_WYT[28]),
      jnp.asarray(_WX[14]), jnp.asarray(_WYT[14]))
    edges = c1_out[3]

    parts = _sc_build()(depths.reshape(_NPIX), edges[:, :16])

    mats = []
    for idx, ((c, h), o) in enumerate(zip(_LAYERS, c1_out[:3])):
        f = (fmap1, fmap2, fmap3)[idx]
        a = f.reshape(2, c, h * h)
        bm = o.reshape(2, h, _D, h).transpose(0, 1, 3, 2).reshape(2, h * h, _D)
        mats += [a, bm]

    return pl.pallas_call(
        _c2_body,
        out_shape=jax.ShapeDtypeStruct((3, 384, _D), jnp.float32),
    )(*mats, parts)
